# baseline (speedup 1.0000x reference)
import functools

import jax
import jax.numpy as jnp
from jax.experimental import pallas as pl
from jax.experimental.pallas import tpu as pltpu


def _se_fused_kernel(x_ref, z_ref, w1t_ref, b1r_ref, w2t_ref, b2r_ref, o_ref,
                     *, inv_hw):
    pooled = jnp.sum(x_ref[...].astype(jnp.float32), axis=(2, 3)) * inv_hw
    h = jnp.dot(pooled, w1t_ref[...],
                preferred_element_type=jnp.float32) + b1r_ref[...]
    h = h * jax.nn.sigmoid(h)
    y = jnp.dot(h, w2t_ref[...],
                preferred_element_type=jnp.float32) + b2r_ref[...]
    s = jax.nn.sigmoid(y)
    s4 = s[:, :, None, None]
    o_ref[...] = (s4 * z_ref[...].astype(jnp.float32)).astype(o_ref.dtype)


def kernel(x, z, w1, b1, w2, b2):
    B, inp, H, W = x.shape
    Bz, oup, Hz, Wz = z.shape
    assert B == Bz
    sq = w1.shape[0]

    w1t = w1.astype(jnp.float32).T
    w2t = w2.astype(jnp.float32).T
    b1r = b1.astype(jnp.float32).reshape(1, sq)
    b2r = b2.astype(jnp.float32).reshape(1, oup)

    return pl.pallas_call(
        functools.partial(_se_fused_kernel, inv_hw=float(1.0 / (H * W))),
        out_shape=jax.ShapeDtypeStruct((B, oup, Hz, Wz), z.dtype),
        grid=(B,),
        in_specs=[
            pl.BlockSpec((1, inp, H, W), lambda b: (b, 0, 0, 0)),
            pl.BlockSpec((1, oup, Hz, Wz), lambda b: (b, 0, 0, 0)),
            pl.BlockSpec((inp, sq), lambda b: (0, 0)),
            pl.BlockSpec((1, sq), lambda b: (0, 0)),
            pl.BlockSpec((sq, oup), lambda b: (0, 0)),
            pl.BlockSpec((1, oup), lambda b: (0, 0)),
        ],
        out_specs=pl.BlockSpec((1, oup, Hz, Wz), lambda b: (b, 0, 0, 0)),
        compiler_params=pltpu.CompilerParams(
            dimension_semantics=("arbitrary",),
            vmem_limit_bytes=56 * 1024 * 1024),
    )(x, z, w1t, b1r, w2t, b2r)
